# baseline (speedup 1.0000x reference)
import functools
import jax
import jax.numpy as jnp
from jax import lax
from jax.experimental import pallas as pl
from jax.experimental.pallas import tpu as pltpu
from jax.experimental.pallas import tpu_sc as plsc

D_MODEL = 64
SCALE = 8.0
NC = 2
NS = 16
NW = NC * NS
LANES = 16

TOT = 4096 * 200
NPW = TOT // NW
CHUNK = 128
NCHUNKS = NPW // CHUNK

_mesh = plsc.VectorSubcoreMesh(core_axis_name="c", subcore_axis_name="s")


@functools.partial(
    pl.kernel,
    out_type=jax.ShapeDtypeStruct((TOT, D_MODEL), jnp.float32),
    mesh=_mesh,
    scratch_types=[
        pltpu.VMEM((CHUNK,), jnp.int32),
        pltpu.VMEM((CHUNK, D_MODEL), jnp.float32),
        pltpu.SemaphoreType.DMA,
    ],
    compiler_params=pltpu.CompilerParams(use_tc_tiling_on_sc=False),
)
def _embed(ids_hbm, table_hbm, out_hbm, idx_v, rows_v, sem):
    wid = lax.axis_index("s") * NC + lax.axis_index("c")
    base0 = wid * NPW

    @pl.loop(0, NCHUNKS)
    def _chunks(g):
        base = base0 + g * CHUNK
        pltpu.sync_copy(ids_hbm.at[pl.ds(base, CHUNK)], idx_v)
        pltpu.async_copy(table_hbm.at[idx_v], rows_v, sem).wait()

        @pl.loop(0, CHUNK)
        def _scale(r):
            for c in range(D_MODEL // LANES):
                sl = pl.ds(c * LANES, LANES)
                rows_v[r, sl] = rows_v[r, sl] * SCALE

        pltpu.sync_copy(rows_v, out_hbm.at[pl.ds(base, CHUNK)])


def kernel(input_ids, weight):
    ids = input_ids.reshape(-1)
    out = _embed(ids, weight)
    return out.reshape(*input_ids.shape, D_MODEL)


# speedup vs baseline: 1.0089x; 1.0089x over previous
import functools
import jax
import jax.numpy as jnp
from jax import lax
from jax.experimental import pallas as pl
from jax.experimental.pallas import tpu as pltpu
from jax.experimental.pallas import tpu_sc as plsc

D_MODEL = 64
SCALE = 8.0
NC = 2
NS = 16
NW = NC * NS
LANES = 16

TOT = 4096 * 200
NPW = TOT // NW
CHUNK = 128
NCHUNKS = NPW // CHUNK
G2 = NCHUNKS // 2

_mesh = plsc.VectorSubcoreMesh(core_axis_name="c", subcore_axis_name="s")


@functools.partial(
    pl.kernel,
    out_type=jax.ShapeDtypeStruct((TOT, D_MODEL), jnp.float32),
    mesh=_mesh,
    scratch_types=[
        pltpu.VMEM((NPW,), jnp.int32),
        pltpu.VMEM((CHUNK, D_MODEL), jnp.float32),
        pltpu.VMEM((CHUNK, D_MODEL), jnp.float32),
        pltpu.VMEM((CHUNK, D_MODEL), jnp.float32),
        pltpu.VMEM((CHUNK, D_MODEL), jnp.float32),
        pltpu.SemaphoreType.DMA,
        pltpu.SemaphoreType.DMA,
        pltpu.SemaphoreType.DMA,
        pltpu.SemaphoreType.DMA,
    ],
    compiler_params=pltpu.CompilerParams(use_tc_tiling_on_sc=False),
)
def _embed(ids_hbm, table_hbm, out_hbm, idx_v, in0, in1, ou0, ou1,
           gs0, gs1, ws0, ws1):
    inb = (in0, in1)
    oub = (ou0, ou1)
    gsem = (gs0, gs1)
    wsem = (ws0, ws1)

    wid = lax.axis_index("s") * NC + lax.axis_index("c")
    base0 = wid * NPW

    pltpu.sync_copy(ids_hbm.at[pl.ds(base0, NPW)], idx_v)

    for b in range(2):
        pltpu.async_copy(
            table_hbm.at[idx_v.at[pl.ds(b * CHUNK, CHUNK)]], inb[b], gsem[b])

    @pl.loop(0, G2)
    def _outer(o):
        for b in range(2):
            g = o * 2 + b
            off = g * CHUNK

            pltpu.make_async_copy(
                table_hbm.at[idx_v.at[pl.ds(off, CHUNK)]], inb[b],
                gsem[b]).wait()

            @pl.when(o > 0)
            def _():
                pltpu.make_async_copy(
                    oub[b],
                    out_hbm.at[pl.ds(base0 + off - 2 * CHUNK, CHUNK)],
                    wsem[b]).wait()

            @pl.loop(0, CHUNK, unroll=8)
            def _scale(r):
                for c in range(D_MODEL // LANES):
                    sl = pl.ds(c * LANES, LANES)
                    oub[b][r, sl] = inb[b][r, sl] * SCALE

            @pl.when(o < G2 - 1)
            def _():
                pltpu.async_copy(
                    table_hbm.at[idx_v.at[pl.ds(off + 2 * CHUNK, CHUNK)]],
                    inb[b], gsem[b])

            pltpu.async_copy(
                oub[b], out_hbm.at[pl.ds(base0 + off, CHUNK)], wsem[b])

    for b in range(2):
        off = (NCHUNKS - 2 + b) * CHUNK
        pltpu.make_async_copy(
            oub[b], out_hbm.at[pl.ds(base0 + off, CHUNK)], wsem[b]).wait()


def kernel(input_ids, weight):
    ids = input_ids.reshape(-1)
    out = _embed(ids, weight)
    return out.reshape(*input_ids.shape, D_MODEL)
